# baseline (speedup 1.0000x reference)
import jax
import jax.numpy as jnp
import numpy as np
from jax.experimental import pallas as pl

EMBED_DIM = 36
OUT_DIMS = [72, 144, 288, 576]
GROUP_NUMS = [1024, 512, 256, 128]
K_NEIGHBORS = 40
ALPHA = 1000.0
BETA = 100.0


def _index_points(points, idx):
    B = points.shape[0]
    bidx = jnp.arange(B).reshape((B,) + (1,) * (idx.ndim - 1))
    return points[bidx, idx]


def _square_distance(src, dst):
    dist = -2.0 * jnp.matmul(src, jnp.transpose(dst, (0, 2, 1)))
    dist = dist + jnp.sum(src ** 2, -1)[:, :, None]
    dist = dist + jnp.sum(dst ** 2, -1)[:, None, :]
    return dist


def _bn(xv, g, b, axes):
    m = jnp.mean(xv, axis=axes, keepdims=True)
    v = jnp.var(xv, axis=axes, keepdims=True)
    xh = (xv - m) / jnp.sqrt(v + 1e-5)
    shape = [1] * xv.ndim
    shape[1] = xv.shape[1]
    return xh * g.reshape(shape) + b.reshape(shape)


def _maxk_pallas(h):
    B, C, G, K = h.shape

    def body(h_ref, o_ref):
        o_ref[...] = jnp.max(h_ref[...], axis=-1)

    Cb, Gb = 72, 128
    return pl.pallas_call(
        body,
        grid=(B, C // Cb, G // Gb),
        in_specs=[pl.BlockSpec((1, Cb, Gb, K), lambda b, c, g: (b, c, g, 0))],
        out_specs=pl.BlockSpec((1, Cb, Gb), lambda b, c, g: (b, c, g)),
        out_shape=jax.ShapeDtypeStruct((B, C, G), h.dtype),
    )(h)


def kernel(xyz, x, params):
    feat = jnp.einsum('oc,bcn->bon', params['w0'], x)
    feat = jax.nn.relu(_bn(feat, params['g0'], params['b0'], (0, 2)))
    cur_xyz, cur_x = xyz, feat
    idx_key = jax.random.key(42)
    for i in range(4):
        B, N = cur_xyz.shape[0], cur_xyz.shape[1]
        G, C_out, K = GROUP_NUMS[i], OUT_DIMS[i], K_NEIGHBORS
        fps_idx = jax.random.randint(jax.random.fold_in(idx_key, i), (B, G), 0, N)
        pts_feat = jnp.transpose(cur_x, (0, 2, 1))
        lc_xyz = _index_points(cur_xyz, fps_idx)
        lc_x = _index_points(pts_feat, fps_idx)
        sqrd = _square_distance(lc_xyz, cur_xyz)
        _, knn_idx = jax.lax.top_k(-sqrd, K)
        knn_xyz = _index_points(cur_xyz, knn_idx)
        knn_x = _index_points(pts_feat, knn_idx)
        mean_xyz = lc_xyz[:, :, None, :]
        std_xyz = jnp.std(knn_xyz - mean_xyz, ddof=1)
        knn_xyz_n = (knn_xyz - mean_xyz) / (std_xyz + 1e-5)
        knn_xc = jnp.concatenate([knn_x, jnp.repeat(lc_x[:, :, None, :], K, axis=2)], axis=-1)
        knn_xyz_p = jnp.transpose(knn_xyz_n, (0, 3, 1, 2))
        h = jnp.transpose(knn_xc, (0, 3, 1, 2))
        fd = C_out // 6
        feat_range = jnp.arange(fd, dtype=jnp.float32)
        dim_embed = jnp.power(ALPHA, feat_range / fd)
        div_embed = BETA * knn_xyz_p[..., None] / dim_embed
        pe = jnp.concatenate([jnp.sin(div_embed), jnp.cos(div_embed)], -1)
        pe = jnp.transpose(pe, (0, 1, 4, 2, 3)).reshape(B, C_out, G, K)
        h = h + pe
        y = jnp.einsum('oc,bcgk->bogk', params['w1_%d' % i], h) + params['b1_%d' % i][None, :, None, None]
        y = jax.nn.relu(_bn(y, params['g1_%d' % i], params['be1_%d' % i], (0, 2, 3)))
        y = jnp.einsum('oc,bcgk->bogk', params['w2_%d' % i], y) + params['b2_%d' % i][None, :, None, None]
        y = _bn(y, params['g2_%d' % i], params['be2_%d' % i], (0, 2, 3))
        h = jax.nn.relu(y + h)
        cur_x = _maxk_pallas(h)
        cur_xyz = lc_xyz
    return cur_xyz, cur_x


# speedup vs baseline: 1.5335x; 1.5335x over previous
import functools

import jax
import jax.numpy as jnp
import numpy as np
from jax.experimental import pallas as pl
from jax.experimental.pallas import tpu as pltpu

EMBED_DIM = 36
OUT_DIMS = [72, 144, 288, 576]
GROUP_NUMS = [1024, 512, 256, 128]
K_NEIGHBORS = 40
ALPHA = 1000.0
BETA = 100.0

GT = 32


def _index_points(points, idx):
    B = points.shape[0]
    bidx = jnp.arange(B).reshape((B,) + (1,) * (idx.ndim - 1))
    return points[bidx, idx]


def _square_distance(src, dst):
    dist = -2.0 * jnp.matmul(src, jnp.transpose(dst, (0, 2, 1)))
    dist = dist + jnp.sum(src ** 2, -1)[:, :, None]
    dist = dist + jnp.sum(dst ** 2, -1)[:, None, :]
    return dist


def _pass_b_body(T, knn_ref, lc_ref, xn_ref, w1t_ref, sf_ref, ph_ref,
                 y1_ref, h_ref, st_ref):
    b, t = pl.program_id(0), pl.program_id(1)
    C_in = knn_ref.shape[1]
    K = K_NEIGHBORS

    arg = jax.lax.dot_general(
        xn_ref[...], sf_ref[...], (((1,), (0,)), ((), ())),
        precision=jax.lax.Precision.HIGHEST,
        preferred_element_type=jnp.float32)
    pe = jnp.sin(arg + ph_ref[...])

    lc_exp = jnp.broadcast_to(lc_ref[...][:, None, :], (GT, K, C_in))
    lc_exp = lc_exp.reshape(GT * K, C_in)
    h = jnp.concatenate([knn_ref[...], lc_exp], axis=1) + pe
    h_ref[...] = h

    y1 = jax.lax.dot_general(
        h, w1t_ref[...], (((1,), (0,)), ((), ())),
        precision=jax.lax.Precision.HIGHEST,
        preferred_element_type=jnp.float32)
    y1_ref[...] = y1

    @pl.when(t == 0)
    def _():
        st_ref[...] = jnp.zeros_like(st_ref)

    st_ref[0:1, :] += jnp.sum(y1, axis=0, keepdims=True)
    st_ref[1:2, :] += jnp.sum(y1 * y1, axis=0, keepdims=True)


def _pass_b(knn_rows, lc_rows, xn_rows, w1t, sf, phase):
    R_tot, C_in = knn_rows.shape
    C_out, hd = w1t.shape
    B = 8
    G = lc_rows.shape[0] // B
    T = G // GT
    R = GT * K_NEIGHBORS
    return pl.pallas_call(
        functools.partial(_pass_b_body, T),
        grid=(B, T),
        in_specs=[
            pl.BlockSpec((R, C_in), lambda b, t: (b * T + t, 0)),
            pl.BlockSpec((GT, C_in), lambda b, t: (b * T + t, 0)),
            pl.BlockSpec((R, 8), lambda b, t: (b * T + t, 0)),
            pl.BlockSpec((C_out, hd), lambda b, t: (0, 0)),
            pl.BlockSpec((8, C_out), lambda b, t: (0, 0)),
            pl.BlockSpec((1, C_out), lambda b, t: (0, 0)),
        ],
        out_specs=[
            pl.BlockSpec((R, hd), lambda b, t: (b * T + t, 0)),
            pl.BlockSpec((R, C_out), lambda b, t: (b * T + t, 0)),
            pl.BlockSpec((8, hd), lambda b, t: (b, 0)),
        ],
        out_shape=[
            jax.ShapeDtypeStruct((R_tot, hd), jnp.float32),
            jax.ShapeDtypeStruct((R_tot, C_out), jnp.float32),
            jax.ShapeDtypeStruct((8 * B, hd), jnp.float32),
        ],
        compiler_params=pltpu.CompilerParams(
            dimension_semantics=("parallel", "arbitrary")),
    )(knn_rows, lc_rows, xn_rows, w1t, sf, phase)


def _pass_c_body(a1_ref, c1_ref, y1_ref, w2t_ref, y2_ref, st_ref):
    b, t = pl.program_id(0), pl.program_id(1)
    r1 = jnp.maximum(y1_ref[...] * a1_ref[...] + c1_ref[...], 0.0)
    y2 = jax.lax.dot_general(
        r1, w2t_ref[...], (((1,), (0,)), ((), ())),
        precision=jax.lax.Precision.HIGHEST,
        preferred_element_type=jnp.float32)
    y2_ref[...] = y2

    @pl.when(t == 0)
    def _():
        st_ref[...] = jnp.zeros_like(st_ref)

    st_ref[0:1, :] += jnp.sum(y2, axis=0, keepdims=True)
    st_ref[1:2, :] += jnp.sum(y2 * y2, axis=0, keepdims=True)


def _pass_c(y1_rows, w2t, a1, c1):
    R_tot, hd = y1_rows.shape
    C_out = w2t.shape[1]
    B = 8
    R = GT * K_NEIGHBORS
    T = R_tot // (B * R)
    return pl.pallas_call(
        _pass_c_body,
        grid=(B, T),
        in_specs=[
            pl.BlockSpec((1, hd), lambda b, t: (0, 0)),
            pl.BlockSpec((1, hd), lambda b, t: (0, 0)),
            pl.BlockSpec((R, hd), lambda b, t: (b * T + t, 0)),
            pl.BlockSpec((hd, C_out), lambda b, t: (0, 0)),
        ],
        out_specs=[
            pl.BlockSpec((R, C_out), lambda b, t: (b * T + t, 0)),
            pl.BlockSpec((8, C_out), lambda b, t: (b, 0)),
        ],
        out_shape=[
            jax.ShapeDtypeStruct((R_tot, C_out), jnp.float32),
            jax.ShapeDtypeStruct((8 * B, C_out), jnp.float32),
        ],
        compiler_params=pltpu.CompilerParams(
            dimension_semantics=("parallel", "arbitrary")),
    )(a1, c1, y1_rows, w2t)


def _pass_d_body(a2_ref, c2_ref, y2_ref, h_ref, o_ref):
    C_out = y2_ref.shape[1]
    hout = jnp.maximum(y2_ref[...] * a2_ref[...] + c2_ref[...] + h_ref[...], 0.0)
    hr = hout.reshape(GT, K_NEIGHBORS, C_out)
    o_ref[...] = jnp.max(hr, axis=1)


def _pass_d(y2_rows, h_rows, a2, c2):
    R_tot, C_out = y2_rows.shape
    B = 8
    R = GT * K_NEIGHBORS
    T = R_tot // (B * R)
    return pl.pallas_call(
        _pass_d_body,
        grid=(B, T),
        in_specs=[
            pl.BlockSpec((1, C_out), lambda b, t: (0, 0)),
            pl.BlockSpec((1, C_out), lambda b, t: (0, 0)),
            pl.BlockSpec((R, C_out), lambda b, t: (b * T + t, 0)),
            pl.BlockSpec((R, C_out), lambda b, t: (b * T + t, 0)),
        ],
        out_specs=pl.BlockSpec((GT, C_out), lambda b, t: (b * T + t, 0)),
        out_shape=jax.ShapeDtypeStruct((R_tot // K_NEIGHBORS, C_out), jnp.float32),
        compiler_params=pltpu.CompilerParams(
            dimension_semantics=("parallel", "parallel")),
    )(a2, c2, y2_rows, h_rows)


def _pe_consts(C_out):
    fd = C_out // 6
    freq = BETA / np.power(ALPHA, np.arange(fd, dtype=np.float64) / fd)
    sf = np.zeros((8, C_out), np.float32)
    phase = np.zeros((1, C_out), np.float32)
    for c in range(C_out):
        d = c // (2 * fd)
        t = c % (2 * fd)
        f = t if t < fd else t - fd
        sf[d, c] = freq[f]
        phase[0, c] = 0.0 if t < fd else np.pi / 2.0
    return jnp.asarray(sf), jnp.asarray(phase)


def _bn_affine(stats, gamma, beta, count):
    st = jnp.sum(stats.reshape(-1, 8, stats.shape[1]), axis=0)
    s, ss = st[0], st[1]
    mean = s / count
    var = ss / count - mean * mean
    a = gamma / jnp.sqrt(var + 1e-5)
    c = beta - mean * a
    return a[None, :], c[None, :]


def kernel(xyz, x, params):
    B, N = xyz.shape[0], xyz.shape[1]
    K = K_NEIGHBORS

    feat = jnp.einsum('oc,bcn->bon', params['w0'], x)
    m = jnp.mean(feat, axis=(0, 2), keepdims=True)
    v = jnp.var(feat, axis=(0, 2), keepdims=True)
    feat = (feat - m) / jnp.sqrt(v + 1e-5)
    feat = jax.nn.relu(feat * params['g0'][None, :, None]
                       + params['b0'][None, :, None])

    cur_xyz = xyz
    cur_rows = jnp.transpose(feat, (0, 2, 1))
    idx_key = jax.random.key(42)

    for i in range(4):
        Ni = cur_xyz.shape[1]
        G, C_out = GROUP_NUMS[i], OUT_DIMS[i]
        C_in = C_out // 2
        hd = C_out // 2

        fps_idx = jax.random.randint(
            jax.random.fold_in(idx_key, i), (B, G), 0, Ni)
        lc_xyz = _index_points(cur_xyz, fps_idx)
        lc_rows = _index_points(cur_rows, fps_idx)

        dist = _square_distance(lc_xyz, cur_xyz)
        _, knn_idx = jax.lax.top_k(-dist, K)

        knn_xyz = _index_points(cur_xyz, knn_idx)
        knn_rows = _index_points(cur_rows, knn_idx)

        diff = knn_xyz - lc_xyz[:, :, None, :]
        std = jnp.std(diff, ddof=1)
        xn = diff / (std + 1e-5)
        xn_rows = jnp.concatenate(
            [xn, jnp.zeros(xn.shape[:3] + (5,), jnp.float32)], axis=-1)
        xn_rows = xn_rows.reshape(B * G * K, 8)

        sf, phase = _pe_consts(C_out)
        w1t = params['w1_%d' % i].T
        w2t = params['w2_%d' % i].T

        y1_rows, h_rows, st1 = _pass_b(
            knn_rows.reshape(B * G * K, C_in),
            lc_rows.reshape(B * G, C_in),
            xn_rows, w1t, sf, phase)

        cnt = float(B * G * K)
        a1, c1 = _bn_affine(st1, params['g1_%d' % i], params['be1_%d' % i], cnt)
        y2_rows, st2 = _pass_c(y1_rows, w2t, a1, c1)
        a2, c2 = _bn_affine(st2, params['g2_%d' % i], params['be2_%d' % i], cnt)
        new_rows = _pass_d(y2_rows, h_rows, a2, c2)

        cur_rows = new_rows.reshape(B, G, C_out)
        cur_xyz = lc_xyz

    return cur_xyz, jnp.transpose(cur_rows, (0, 2, 1))
